# Initial kernel scaffold; baseline (speedup 1.0000x reference)
import functools

import jax
import jax.numpy as jnp
from jax import lax
from jax.experimental import pallas as pl
from jax.experimental.pallas import tpu as pltpu
from jax.experimental.pallas import tpu_sc as plsc

_NC = 2
_NS = 16
_NW = _NC * _NS
_K = 80


def _sc_aggregate(table, srcs, dsts, n_rows, width):
  nw, ch, k = srcs.shape
  with_deg = width == 128
  npt = n_rows // _NS
  mesh = plsc.VectorSubcoreMesh(core_axis_name="c", subcore_axis_name="s")

  out_type = [jax.ShapeDtypeStruct((_NC, n_rows, width), jnp.float32)]
  scratch = [
      pltpu.VMEM((ch, k), jnp.int32),
      pltpu.VMEM((ch, k), jnp.int32),
      pltpu.VMEM((k, width), jnp.float32),
      pltpu.VMEM_SHARED((n_rows, width), jnp.float32),
  ]
  if with_deg:
    out_type.append(jax.ShapeDtypeStruct((_NC, n_rows, 16), jnp.float32))
    scratch.append(pltpu.VMEM((k, 16), jnp.float32))
    scratch.append(pltpu.VMEM_SHARED((n_rows, 16), jnp.float32))

  def body(table_hbm, src_hbm, dst_hbm, *rest):
    if with_deg:
      sums_out, deg_out, src_v, dst_v, rows_v, sums_sh, ones_v, deg_sh = rest
    else:
      sums_out, src_v, dst_v, rows_v, sums_sh = rest
    c = lax.axis_index("c")
    s = lax.axis_index("s")
    wid = c * _NS + s

    def zero_buf(buf, w):
      def zrow(i, _):
        for cc in range(w // 16):
          buf[i, pl.ds(cc * 16, 16)] = jnp.zeros((16,), jnp.float32)
        return 0
      lax.fori_loop(0, k, zrow, 0)

    zero_buf(rows_v, width)
    base = s * npt
    nfull = npt // k
    rem = npt - nfull * k
    for t in range(nfull):
      pltpu.sync_copy(rows_v, sums_sh.at[pl.ds(base + t * k, k)])
    if rem:
      pltpu.sync_copy(rows_v.at[pl.ds(0, rem)],
                      sums_sh.at[pl.ds(base + nfull * k, rem)])
    if with_deg:
      zero_buf(ones_v, 16)
      for t in range(nfull):
        pltpu.sync_copy(ones_v, deg_sh.at[pl.ds(base + t * k, k)])
      if rem:
        pltpu.sync_copy(ones_v.at[pl.ds(0, rem)],
                        deg_sh.at[pl.ds(base + nfull * k, rem)])

      def one_row(i, _):
        ones_v[i, pl.ds(0, 16)] = jnp.ones((16,), jnp.float32)
        return 0
      lax.fori_loop(0, k, one_row, 0)

    pltpu.sync_copy(src_hbm.at[wid], src_v)
    pltpu.sync_copy(dst_hbm.at[wid], dst_v)
    plsc.subcore_barrier()

    def edge_chunk(j, _):
      pltpu.sync_copy(table_hbm.at[src_v.at[j]], rows_v)
      pltpu.sync_copy(rows_v, sums_sh.at[dst_v.at[j]], add=True)
      if with_deg:
        pltpu.sync_copy(ones_v, deg_sh.at[dst_v.at[j]], add=True)
      return 0
    lax.fori_loop(0, ch, edge_chunk, 0)

    plsc.subcore_barrier()
    pltpu.sync_copy(sums_sh.at[pl.ds(base, npt)],
                    sums_out.at[c, pl.ds(base, npt)])
    if with_deg:
      pltpu.sync_copy(deg_sh.at[pl.ds(base, npt)],
                      deg_out.at[c, pl.ds(base, npt)])

  fn = pl.kernel(body, out_type=out_type, mesh=mesh, scratch_types=scratch)
  return fn(table, srcs, dsts)


def _tc_layer1(features, sums_p, deg_p, W_self1, W_neigh1, b1,
               W_self2, W_neigh2, b2):
  n, d = features.shape
  h = W_self1.shape[1]
  c_dim = W_self2.shape[1]
  nb = 8
  r = n // nb

  def body(f_ref, sp_ref, dp_ref, ws1_ref, wn1_ref, b1_ref, ws2_ref,
           wn2_ref, b2_ref, y2_ref, pre2_ref):
    dp = dp_ref[...]
    deg = jnp.clip(dp[0, :, :1] + dp[1, :, :1], 1.0, None)
    sp = sp_ref[...]
    sums1 = sp[0] + sp[1]
    hn1 = jnp.dot(sums1, wn1_ref[...],
                  preferred_element_type=jnp.float32) / deg
    h1 = jnp.maximum(
        jnp.dot(f_ref[...], ws1_ref[...], preferred_element_type=jnp.float32)
        + hn1 + b1_ref[...], 0.0)
    y2_ref[...] = jnp.dot(h1, wn2_ref[...], preferred_element_type=jnp.float32)
    pre2_ref[...] = (
        jnp.dot(h1, ws2_ref[...], preferred_element_type=jnp.float32)
        + b2_ref[...])

  return pl.pallas_call(
      body,
      grid=(nb,),
      in_specs=[
          pl.BlockSpec((r, d), lambda i: (i, 0)),
          pl.BlockSpec((2, r, d), lambda i: (0, i, 0)),
          pl.BlockSpec((2, r, 16), lambda i: (0, i, 0)),
          pl.BlockSpec((d, h), lambda i: (0, 0)),
          pl.BlockSpec((d, h), lambda i: (0, 0)),
          pl.BlockSpec((1, h), lambda i: (0, 0)),
          pl.BlockSpec((h, c_dim), lambda i: (0, 0)),
          pl.BlockSpec((h, c_dim), lambda i: (0, 0)),
          pl.BlockSpec((1, c_dim), lambda i: (0, 0)),
      ],
      out_specs=[
          pl.BlockSpec((r, c_dim), lambda i: (i, 0)),
          pl.BlockSpec((r, c_dim), lambda i: (i, 0)),
      ],
      out_shape=[
          jax.ShapeDtypeStruct((n, c_dim), jnp.float32),
          jax.ShapeDtypeStruct((n, c_dim), jnp.float32),
      ],
  )(features, sums_p, deg_p, W_self1, W_neigh1, b1.reshape(1, h),
    W_self2, W_neigh2, b2.reshape(1, c_dim))


def _tc_layer2(pre2, sums2_p, deg_p):
  n, c_dim = pre2.shape
  nb = 8
  r = n // nb

  def body(pre_ref, q_ref, dp_ref, out_ref):
    dp = dp_ref[...]
    deg = jnp.clip(dp[0, :, :1] + dp[1, :, :1], 1.0, None)
    q = q_ref[...]
    out_ref[...] = pre_ref[...] + (q[0] + q[1]) / deg

  return pl.pallas_call(
      body,
      grid=(nb,),
      in_specs=[
          pl.BlockSpec((r, c_dim), lambda i: (i, 0)),
          pl.BlockSpec((2, r, c_dim), lambda i: (0, i, 0)),
          pl.BlockSpec((2, r, 16), lambda i: (0, i, 0)),
      ],
      out_specs=pl.BlockSpec((r, c_dim), lambda i: (i, 0)),
      out_shape=jax.ShapeDtypeStruct((n, c_dim), jnp.float32),
  )(pre2, sums2_p, deg_p)


@jax.jit
def kernel(features, edge_index, W_self1, W_neigh1, b1, W_self2, W_neigh2,
           b2):
  n = features.shape[0]
  e = edge_index.shape[1]
  ch = e // (_NW * _K)
  srcs = edge_index[0].reshape(_NW, ch, _K)
  dsts = edge_index[1].reshape(_NW, ch, _K)

  sums1_p, deg_p = _sc_aggregate(features, srcs, dsts, n, 128)
  y2, pre2 = _tc_layer1(features, sums1_p, deg_p, W_self1, W_neigh1, b1,
                        W_self2, W_neigh2, b2)
  (sums2_p,) = _sc_aggregate(y2, srcs, dsts, n, 16)
  return _tc_layer2(pre2, sums2_p, deg_p)


# speedup vs baseline: 1.7390x; 1.7390x over previous
import functools

import jax
import jax.numpy as jnp
from jax import lax
from jax.experimental import pallas as pl
from jax.experimental.pallas import tpu as pltpu
from jax.experimental.pallas import tpu_sc as plsc

_NC = 2
_NS = 16
_NW = _NC * _NS
_K = 80


def _sc_aggregate(table, edges, n_rows, width, pattern=None):
  nw, ch, _, k = edges.shape
  with_deg = pattern is not None
  npt = n_rows // _NS
  dpt = npt // 8
  mesh = plsc.VectorSubcoreMesh(core_axis_name="c", subcore_axis_name="s")

  out_type = [jax.ShapeDtypeStruct((_NC, n_rows, width), jnp.float32)]
  scratch = [
      pltpu.VMEM((4, k), jnp.int32),
      pltpu.VMEM((k, width), jnp.float32),
      pltpu.VMEM_SHARED((n_rows, width), jnp.float32),
  ]
  if with_deg:
    out_type.append(jax.ShapeDtypeStruct((_NC, n_rows // 8, 128),
                                         jnp.float32))
    scratch.append(pltpu.VMEM((k, 128), jnp.float32))
    scratch.append(pltpu.VMEM_SHARED((n_rows // 8, 128), jnp.float32))

  def body(table_hbm, edges_hbm, *rest):
    if with_deg:
      pat_hbm, sums_out, deg_out, idx_v, rows_v, sums_sh, sel_v, deg_sh = rest
    else:
      sums_out, idx_v, rows_v, sums_sh = rest
    c = lax.axis_index("c")
    s = lax.axis_index("s")
    wid = c * _NS + s

    def zrow(i, _):
      for cc in range(width // 16):
        rows_v[i, pl.ds(cc * 16, 16)] = jnp.zeros((16,), jnp.float32)
      return 0
    lax.fori_loop(0, k, zrow, 0)
    base = s * npt
    for t in range(npt // k):
      pltpu.sync_copy(rows_v, sums_sh.at[pl.ds(base + t * k, k)])
    if with_deg:
      pltpu.sync_copy(rows_v, deg_sh.at[pl.ds(s * dpt, dpt)])
    plsc.subcore_barrier()

    def edge_chunk(j, _):
      pltpu.sync_copy(edges_hbm.at[wid, j], idx_v)
      pltpu.sync_copy(table_hbm.at[idx_v.at[0]], rows_v)
      pltpu.sync_copy(rows_v, sums_sh.at[idx_v.at[1]], add=True)
      if with_deg:
        pltpu.sync_copy(pat_hbm.at[idx_v.at[3]], sel_v)
        pltpu.sync_copy(sel_v, deg_sh.at[idx_v.at[2]], add=True)
      return 0
    lax.fori_loop(0, ch, edge_chunk, 0)

    plsc.subcore_barrier()
    pltpu.sync_copy(sums_sh.at[pl.ds(base, npt)],
                    sums_out.at[c, pl.ds(base, npt)])
    if with_deg:
      pltpu.sync_copy(deg_sh.at[pl.ds(s * dpt, dpt)],
                      deg_out.at[c, pl.ds(s * dpt, dpt)])

  fn = pl.kernel(body, out_type=out_type, mesh=mesh, scratch_types=scratch)
  if with_deg:
    return fn(table, edges, pattern)
  return fn(table, edges)


def _tc_layer1(features, sums_p, deg_p, W_self1, W_neigh1, b1,
               W_self2, b2):
  n, d = features.shape
  h = W_self1.shape[1]
  c_dim = W_self2.shape[1]
  r = 1024
  nb = (n + r - 1) // r

  def body(f_ref, sp_ref, dp_ref, ws1_ref, wn1_ref, b1_ref, ws2_ref,
           b2_ref, h1_ref, pre2_ref):
    dp = dp_ref[...]
    deg = jnp.maximum(dp[0, :, :1] + dp[1, :, :1], 1.0)
    sp = sp_ref[...]
    sums1 = sp[0] + sp[1]
    hn1 = jnp.dot(sums1, wn1_ref[...],
                  preferred_element_type=jnp.float32) / deg
    h1 = jnp.maximum(
        jnp.dot(f_ref[...], ws1_ref[...], preferred_element_type=jnp.float32)
        + hn1 + b1_ref[...], 0.0)
    h1_ref[...] = h1
    pre2_ref[...] = (
        jnp.dot(h1, ws2_ref[...], preferred_element_type=jnp.float32)
        + b2_ref[...])

  return pl.pallas_call(
      body,
      grid=(nb,),
      in_specs=[
          pl.BlockSpec((r, d), lambda i: (i, 0)),
          pl.BlockSpec((2, r, d), lambda i: (0, i, 0)),
          pl.BlockSpec((2, r, 16), lambda i: (0, i, 0)),
          pl.BlockSpec((d, h), lambda i: (0, 0)),
          pl.BlockSpec((d, h), lambda i: (0, 0)),
          pl.BlockSpec((1, h), lambda i: (0, 0)),
          pl.BlockSpec((h, c_dim), lambda i: (0, 0)),
          pl.BlockSpec((1, c_dim), lambda i: (0, 0)),
      ],
      out_specs=[
          pl.BlockSpec((r, h), lambda i: (i, 0)),
          pl.BlockSpec((r, c_dim), lambda i: (i, 0)),
      ],
      out_shape=[
          jax.ShapeDtypeStruct((n, h), jnp.float32),
          jax.ShapeDtypeStruct((n, c_dim), jnp.float32),
      ],
  )(features, sums_p, deg_p, W_self1, W_neigh1, b1.reshape(1, h),
    W_self2, b2.reshape(1, c_dim))


def _tc_layer2(pre2, sums2_p, deg_p, W_neigh2):
  n, c_dim = pre2.shape
  h = W_neigh2.shape[0]
  r = 1024
  nb = (n + r - 1) // r

  def body(pre_ref, q_ref, dp_ref, wn2_ref, out_ref):
    dp = dp_ref[...]
    deg = jnp.maximum(dp[0, :, :1] + dp[1, :, :1], 1.0)
    q = q_ref[...]
    out_ref[...] = pre_ref[...] + jnp.dot(
        q[0] + q[1], wn2_ref[...], preferred_element_type=jnp.float32) / deg

  return pl.pallas_call(
      body,
      grid=(nb,),
      in_specs=[
          pl.BlockSpec((r, c_dim), lambda i: (i, 0)),
          pl.BlockSpec((2, r, h), lambda i: (0, i, 0)),
          pl.BlockSpec((2, r, 16), lambda i: (0, i, 0)),
          pl.BlockSpec((h, c_dim), lambda i: (0, 0)),
      ],
      out_specs=pl.BlockSpec((r, c_dim), lambda i: (i, 0)),
      out_shape=jax.ShapeDtypeStruct((n, c_dim), jnp.float32),
  )(pre2, sums2_p, deg_p, W_neigh2)


@jax.jit
def kernel(features, edge_index, W_self1, W_neigh1, b1, W_self2, W_neigh2,
           b2):
  n = features.shape[0]
  e = edge_index.shape[1]
  ch = e // (_NW * _K)
  src = edge_index[0]
  dst = edge_index[1]
  edges = jnp.stack([src, dst, dst >> 3, dst & 7]).reshape(
      4, _NW, ch, _K).transpose(1, 2, 0, 3)
  pattern = jnp.kron(jnp.eye(8, dtype=jnp.float32),
                     jnp.ones((1, 16), jnp.float32))
  np_rows = -(-n // (_NS * _K)) * (_NS * _K)

  sums1_p, deg_p = _sc_aggregate(features, edges, np_rows, 128,
                                 pattern=pattern)
  deg_p = deg_p.reshape(_NC, np_rows, 16)
  h1, pre2 = _tc_layer1(features, sums1_p, deg_p, W_self1, W_neigh1, b1,
                        W_self2, b2)
  (sums2_p,) = _sc_aggregate(h1, edges, np_rows, 128)
  return _tc_layer2(pre2, sums2_p, deg_p, W_neigh2)


# speedup vs baseline: 1.8737x; 1.0774x over previous
import functools

import jax
import jax.numpy as jnp
from jax import lax
from jax.experimental import pallas as pl
from jax.experimental.pallas import tpu as pltpu
from jax.experimental.pallas import tpu_sc as plsc

_NC = 2
_NS = 16
_NW = _NC * _NS
_K = 80


def _sc_aggregate(table, edges, n_rows, width, pattern=None):
  nw, ch, _, k = edges.shape
  with_deg = pattern is not None
  npt = n_rows // _NS
  dpt = npt // 8
  blk = 25
  nblk = ch // blk
  mesh = plsc.VectorSubcoreMesh(core_axis_name="c", subcore_axis_name="s")

  out_type = [jax.ShapeDtypeStruct((_NC, n_rows, width), jnp.float32)]
  scratch = [
      pltpu.VMEM((blk, 4, k), jnp.int32),
      pltpu.VMEM((2, k, width), jnp.float32),
      pltpu.VMEM_SHARED((n_rows, width), jnp.float32),
      pltpu.SemaphoreType.DMA,
      pltpu.SemaphoreType.DMA,
  ]
  if with_deg:
    out_type.append(jax.ShapeDtypeStruct((_NC, n_rows // 8, 128),
                                         jnp.float32))
    scratch.append(pltpu.VMEM_SHARED((n_rows // 8, 128), jnp.float32))

  def body(table_hbm, edges_hbm, *rest):
    if with_deg:
      (pat_hbm, sums_out, deg_out, idx_blk, rows_v, sums_sh, sem_g, sem_s,
       deg_sh) = rest
    else:
      sums_out, idx_blk, rows_v, sums_sh, sem_g, sem_s = rest
    c = lax.axis_index("c")
    s = lax.axis_index("s")
    wid = c * _NS + s

    def zrow(i, _):
      for cc in range(width // 16):
        rows_v[0, i, pl.ds(cc * 16, 16)] = jnp.zeros((16,), jnp.float32)
      return 0
    lax.fori_loop(0, k, zrow, 0)
    base = s * npt
    for t in range(npt // k):
      pltpu.sync_copy(rows_v.at[0], sums_sh.at[pl.ds(base + t * k, k)])
    if with_deg:
      pltpu.sync_copy(rows_v.at[0], deg_sh.at[pl.ds(s * dpt, dpt)])
    plsc.subcore_barrier()

    def run_pipe(mk_g, mk_s):
      mk_g(0, 0).start()

      def step(j, _):
        @pl.when(j >= 1)
        def _():
          mk_s(j - 1, (j - 1) % 2).wait()

        @pl.when(j < blk - 1)
        def _():
          mk_g(j + 1, (j + 1) % 2).start()
        mk_g(j, j % 2).wait()
        mk_s(j, j % 2).start(add=True)
        return 0
      lax.fori_loop(0, blk, step, 0)
      mk_s(blk - 1, (blk - 1) % 2).wait()

    def mk_sum_g(j, b):
      return pltpu.make_async_copy(
          table_hbm.at[idx_blk.at[j, 0]], rows_v.at[b], sem_g)

    def mk_sum_s(j, b):
      return pltpu.make_async_copy(
          rows_v.at[b], sums_sh.at[idx_blk.at[j, 1]], sem_s)

    def mk_deg_g(j, b):
      return pltpu.make_async_copy(
          pat_hbm.at[idx_blk.at[j, 3]], rows_v.at[b], sem_g)

    def mk_deg_s(j, b):
      return pltpu.make_async_copy(
          rows_v.at[b], deg_sh.at[idx_blk.at[j, 2]], sem_s)

    def block_step(bi, _):
      pltpu.sync_copy(edges_hbm.at[wid, pl.ds(bi * blk, blk)], idx_blk)
      run_pipe(mk_sum_g, mk_sum_s)
      if with_deg:
        run_pipe(mk_deg_g, mk_deg_s)
      return 0
    lax.fori_loop(0, nblk, block_step, 0)

    plsc.subcore_barrier()
    pltpu.sync_copy(sums_sh.at[pl.ds(base, npt)],
                    sums_out.at[c, pl.ds(base, npt)])
    if with_deg:
      pltpu.sync_copy(deg_sh.at[pl.ds(s * dpt, dpt)],
                      deg_out.at[c, pl.ds(s * dpt, dpt)])

  fn = pl.kernel(body, out_type=out_type, mesh=mesh, scratch_types=scratch)
  if with_deg:
    return fn(table, edges, pattern)
  return fn(table, edges)


def _tc_layer1(features, sums_p, deg_p, W_self1, W_neigh1, b1,
               W_self2, b2):
  n, d = features.shape
  h = W_self1.shape[1]
  c_dim = W_self2.shape[1]
  r = 1024
  nb = (n + r - 1) // r

  def body(f_ref, sp_ref, dp_ref, ws1_ref, wn1_ref, b1_ref, ws2_ref,
           b2_ref, h1_ref, pre2_ref):
    dp = dp_ref[...]
    deg = jnp.maximum(dp[0, :, :1] + dp[1, :, :1], 1.0)
    sp = sp_ref[...]
    sums1 = sp[0] + sp[1]
    hn1 = jnp.dot(sums1, wn1_ref[...],
                  preferred_element_type=jnp.float32) / deg
    h1 = jnp.maximum(
        jnp.dot(f_ref[...], ws1_ref[...], preferred_element_type=jnp.float32)
        + hn1 + b1_ref[...], 0.0)
    h1_ref[...] = h1
    pre2_ref[...] = (
        jnp.dot(h1, ws2_ref[...], preferred_element_type=jnp.float32)
        + b2_ref[...])

  return pl.pallas_call(
      body,
      grid=(nb,),
      in_specs=[
          pl.BlockSpec((r, d), lambda i: (i, 0)),
          pl.BlockSpec((2, r, d), lambda i: (0, i, 0)),
          pl.BlockSpec((2, r, 16), lambda i: (0, i, 0)),
          pl.BlockSpec((d, h), lambda i: (0, 0)),
          pl.BlockSpec((d, h), lambda i: (0, 0)),
          pl.BlockSpec((1, h), lambda i: (0, 0)),
          pl.BlockSpec((h, c_dim), lambda i: (0, 0)),
          pl.BlockSpec((1, c_dim), lambda i: (0, 0)),
      ],
      out_specs=[
          pl.BlockSpec((r, h), lambda i: (i, 0)),
          pl.BlockSpec((r, c_dim), lambda i: (i, 0)),
      ],
      out_shape=[
          jax.ShapeDtypeStruct((n, h), jnp.float32),
          jax.ShapeDtypeStruct((n, c_dim), jnp.float32),
      ],
  )(features, sums_p, deg_p, W_self1, W_neigh1, b1.reshape(1, h),
    W_self2, b2.reshape(1, c_dim))


def _tc_layer2(pre2, sums2_p, deg_p, W_neigh2):
  n, c_dim = pre2.shape
  h = W_neigh2.shape[0]
  r = 1024
  nb = (n + r - 1) // r

  def body(pre_ref, q_ref, dp_ref, wn2_ref, out_ref):
    dp = dp_ref[...]
    deg = jnp.maximum(dp[0, :, :1] + dp[1, :, :1], 1.0)
    q = q_ref[...]
    out_ref[...] = pre_ref[...] + jnp.dot(
        q[0] + q[1], wn2_ref[...], preferred_element_type=jnp.float32) / deg

  return pl.pallas_call(
      body,
      grid=(nb,),
      in_specs=[
          pl.BlockSpec((r, c_dim), lambda i: (i, 0)),
          pl.BlockSpec((2, r, h), lambda i: (0, i, 0)),
          pl.BlockSpec((2, r, 16), lambda i: (0, i, 0)),
          pl.BlockSpec((h, c_dim), lambda i: (0, 0)),
      ],
      out_specs=pl.BlockSpec((r, c_dim), lambda i: (i, 0)),
      out_shape=jax.ShapeDtypeStruct((n, c_dim), jnp.float32),
  )(pre2, sums2_p, deg_p, W_neigh2)


@jax.jit
def kernel(features, edge_index, W_self1, W_neigh1, b1, W_self2, W_neigh2,
           b2):
  n = features.shape[0]
  e = edge_index.shape[1]
  ch = e // (_NW * _K)
  src = edge_index[0]
  dst = edge_index[1]
  edges = jnp.stack([src, dst, dst >> 3, dst & 7]).reshape(
      4, _NW, ch, _K).transpose(1, 2, 0, 3)
  pattern = jnp.kron(jnp.eye(8, dtype=jnp.float32),
                     jnp.ones((1, 16), jnp.float32))
  np_rows = -(-n // (_NS * _K)) * (_NS * _K)

  sums1_p, deg_p = _sc_aggregate(features, edges, np_rows, 128,
                                 pattern=pattern)
  deg_p = deg_p.reshape(_NC, np_rows, 16)
  h1, pre2 = _tc_layer1(features, sums1_p, deg_p, W_self1, W_neigh1, b1,
                        W_self2, b2)
  (sums2_p,) = _sc_aggregate(h1, edges, np_rows, 128)
  return _tc_layer2(pre2, sums2_p, deg_p, W_neigh2)


# speedup vs baseline: 7.6662x; 4.0916x over previous
import functools

import jax
import jax.numpy as jnp
from jax import lax
from jax.experimental import pallas as pl
from jax.experimental.pallas import tpu as pltpu
from jax.experimental.pallas import tpu_sc as plsc

_NC = 2
_NS = 16
_NW = _NC * _NS
_K = 80


def _sc_aggregate(table, edges, n_rows, width, pattern=None):
  nw, ch, _, k = edges.shape
  with_deg = pattern is not None
  npt = n_rows // _NS
  dpt = npt // 8
  blk = 25
  nblk = ch // blk
  mesh = plsc.VectorSubcoreMesh(core_axis_name="c", subcore_axis_name="s")

  out_type = [jax.ShapeDtypeStruct((_NC, n_rows, width), jnp.float32)]
  scratch = [
      pltpu.VMEM((blk, 4, k), jnp.int32),
      pltpu.VMEM((2, k, width), jnp.float32),
      pltpu.VMEM_SHARED((n_rows, width), jnp.float32),
      pltpu.SemaphoreType.DMA,
      pltpu.SemaphoreType.DMA,
  ]
  if with_deg:
    out_type.append(jax.ShapeDtypeStruct((_NC, n_rows // 8, 128),
                                         jnp.float32))
    scratch.append(pltpu.VMEM_SHARED((n_rows // 8, 128), jnp.float32))

  def body(table_hbm, edges_hbm, *rest):
    if with_deg:
      (pat_hbm, sums_out, deg_out, idx_blk, rows_v, sums_sh, sem_g, sem_s,
       deg_sh) = rest
    else:
      sums_out, idx_blk, rows_v, sums_sh, sem_g, sem_s = rest
    c = lax.axis_index("c")
    s = lax.axis_index("s")
    wid = c * _NS + s

    def zrow(i, _):
      for cc in range(width // 16):
        rows_v[0, i, pl.ds(cc * 16, 16)] = jnp.zeros((16,), jnp.float32)
      return 0
    lax.fori_loop(0, k, zrow, 0)
    base = s * npt
    for t in range(npt // k):
      pltpu.sync_copy(rows_v.at[0], sums_sh.at[pl.ds(base + t * k, k)])
    if with_deg:
      pltpu.sync_copy(rows_v.at[0], deg_sh.at[pl.ds(s * dpt, dpt)])
    plsc.subcore_barrier()

    def run_pipe(mk_g, mk_s):
      mk_g(0, 0).start()

      def step(j, _):
        @pl.when(j >= 1)
        def _():
          mk_s(j - 1, (j - 1) % 2).wait()

        @pl.when(j < blk - 1)
        def _():
          mk_g(j + 1, (j + 1) % 2).start()
        mk_g(j, j % 2).wait()
        mk_s(j, j % 2).start(add=True)
        return 0
      lax.fori_loop(0, blk, step, 0)
      mk_s(blk - 1, (blk - 1) % 2).wait()

    def mk_sum_g(j, b):
      return pltpu.make_async_copy(
          table_hbm.at[idx_blk.at[j, 0]], rows_v.at[b], sem_g)

    def mk_sum_s(j, b):
      return pltpu.make_async_copy(
          rows_v.at[b], sums_sh.at[idx_blk.at[j, 1]], sem_s)

    def mk_deg_g(j, b):
      return pltpu.make_async_copy(
          pat_hbm.at[idx_blk.at[j, 3]], rows_v.at[b], sem_g)

    def mk_deg_s(j, b):
      return pltpu.make_async_copy(
          rows_v.at[b], deg_sh.at[idx_blk.at[j, 2]], sem_s)

    def block_step(bi, _):
      pltpu.sync_copy(edges_hbm.at[wid, pl.ds(bi * blk, blk)], idx_blk)
      run_pipe(mk_sum_g, mk_sum_s)
      if with_deg:
        run_pipe(mk_deg_g, mk_deg_s)
      return 0
    lax.fori_loop(0, nblk, block_step, 0)

    plsc.subcore_barrier()
    pltpu.sync_copy(sums_sh.at[pl.ds(base, npt)],
                    sums_out.at[c, pl.ds(base, npt)])
    if with_deg:
      pltpu.sync_copy(deg_sh.at[pl.ds(s * dpt, dpt)],
                      deg_out.at[c, pl.ds(s * dpt, dpt)])

  fn = pl.kernel(body, out_type=out_type, mesh=mesh, scratch_types=scratch)
  if with_deg:
    return fn(table, edges, pattern)
  return fn(table, edges)


def _tc_layer1(features, sums_p, deg_p, W_self1, W_neigh1, b1,
               W_self2, b2):
  n, d = features.shape
  h = W_self1.shape[1]
  c_dim = W_self2.shape[1]
  r = 1024
  nb = (n + r - 1) // r

  def body(f_ref, sp_ref, dp_ref, ws1_ref, wn1_ref, b1_ref, ws2_ref,
           b2_ref, h1_ref, pre2_ref):
    dp = dp_ref[...]
    deg = jnp.maximum(dp[0, :, :1] + dp[1, :, :1], 1.0)
    sp = sp_ref[...]
    sums1 = sp[0] + sp[1]
    hn1 = jnp.dot(sums1, wn1_ref[...],
                  preferred_element_type=jnp.float32) / deg
    h1 = jnp.maximum(
        jnp.dot(f_ref[...], ws1_ref[...], preferred_element_type=jnp.float32)
        + hn1 + b1_ref[...], 0.0)
    h1_ref[...] = h1
    pre2_ref[...] = (
        jnp.dot(h1, ws2_ref[...], preferred_element_type=jnp.float32)
        + b2_ref[...])

  return pl.pallas_call(
      body,
      grid=(nb,),
      in_specs=[
          pl.BlockSpec((r, d), lambda i: (i, 0)),
          pl.BlockSpec((2, r, d), lambda i: (0, i, 0)),
          pl.BlockSpec((2, r, 16), lambda i: (0, i, 0)),
          pl.BlockSpec((d, h), lambda i: (0, 0)),
          pl.BlockSpec((d, h), lambda i: (0, 0)),
          pl.BlockSpec((1, h), lambda i: (0, 0)),
          pl.BlockSpec((h, c_dim), lambda i: (0, 0)),
          pl.BlockSpec((1, c_dim), lambda i: (0, 0)),
      ],
      out_specs=[
          pl.BlockSpec((r, h), lambda i: (i, 0)),
          pl.BlockSpec((r, c_dim), lambda i: (i, 0)),
      ],
      out_shape=[
          jax.ShapeDtypeStruct((n, h), jnp.float32),
          jax.ShapeDtypeStruct((n, c_dim), jnp.float32),
      ],
  )(features, sums_p, deg_p, W_self1, W_neigh1, b1.reshape(1, h),
    W_self2, b2.reshape(1, c_dim))


def _tc_layer2(pre2, sums2_p, deg_p, W_neigh2):
  n, c_dim = pre2.shape
  h = W_neigh2.shape[0]
  r = 1024
  nb = (n + r - 1) // r

  def body(pre_ref, q_ref, dp_ref, wn2_ref, out_ref):
    dp = dp_ref[...]
    deg = jnp.maximum(dp[0, :, :1] + dp[1, :, :1], 1.0)
    q = q_ref[...]
    out_ref[...] = pre_ref[...] + jnp.dot(
        q[0] + q[1], wn2_ref[...], preferred_element_type=jnp.float32) / deg

  return pl.pallas_call(
      body,
      grid=(nb,),
      in_specs=[
          pl.BlockSpec((r, c_dim), lambda i: (i, 0)),
          pl.BlockSpec((2, r, h), lambda i: (0, i, 0)),
          pl.BlockSpec((2, r, 16), lambda i: (0, i, 0)),
          pl.BlockSpec((h, c_dim), lambda i: (0, 0)),
      ],
      out_specs=pl.BlockSpec((r, c_dim), lambda i: (i, 0)),
      out_shape=jax.ShapeDtypeStruct((n, c_dim), jnp.float32),
  )(pre2, sums2_p, deg_p, W_neigh2)


@jax.jit
def kernel(features, edge_index, W_self1, W_neigh1, b1, W_self2, W_neigh2,
           b2):
  n = features.shape[0]
  e = edge_index.shape[1]
  ch = e // (_NW * _K)
  src = edge_index[0]
  dst = edge_index[1]
  pattern = jnp.tile(
      jnp.kron(jnp.eye(8, dtype=jnp.float32),
               jnp.ones((1, 16), jnp.float32)), (256, 1))
  pat_idx = (dst & 7) + 8 * (jnp.arange(e, dtype=jnp.int32) & 255)
  edges = jnp.stack([src, dst, dst >> 3, pat_idx]).reshape(
      4, _NW, ch, _K).transpose(1, 2, 0, 3)
  np_rows = -(-n // (_NS * _K)) * (_NS * _K)

  sums1_p, deg_p = _sc_aggregate(features, edges, np_rows, 128,
                                 pattern=pattern)
  deg_p = deg_p.reshape(_NC, np_rows, 16)
  h1, pre2 = _tc_layer1(features, sums1_p, deg_p, W_self1, W_neigh1, b1,
                        W_self2, b2)
  (sums2_p,) = _sc_aggregate(h1, edges, np_rows, 128)
  return _tc_layer2(pre2, sums2_p, deg_p, W_neigh2)


# speedup vs baseline: 7.8628x; 1.0257x over previous
import functools

import jax
import jax.numpy as jnp
from jax import lax
from jax.experimental import pallas as pl
from jax.experimental.pallas import tpu as pltpu
from jax.experimental.pallas import tpu_sc as plsc

_NC = 2
_NS = 16
_NW = _NC * _NS
_K = 80
_BLK = 25


def _sc_aggregate(table, n_rows, srcs, dsts, divs=None, pats=None,
                  pattern=None):
  width = table.shape[1]
  nw, ch, _, k = srcs.shape
  with_deg = pattern is not None
  npt = n_rows // _NS
  dpt = npt // 8
  nblk = ch // _BLK
  mesh = plsc.VectorSubcoreMesh(core_axis_name="c", subcore_axis_name="s")

  out_type = [jax.ShapeDtypeStruct((_NC, n_rows, width), jnp.float32)]
  scratch = [
      pltpu.VMEM((_BLK, 1, k), jnp.int32),
      pltpu.VMEM((_BLK, 1, k), jnp.int32),
      pltpu.VMEM((2, k, width), jnp.float32),
      pltpu.VMEM_SHARED((n_rows, width), jnp.float32),
      pltpu.SemaphoreType.DMA,
      pltpu.SemaphoreType.DMA,
  ]
  if with_deg:
    out_type.append(jax.ShapeDtypeStruct((_NC, n_rows // 8, 128),
                                         jnp.float32))
    scratch.append(pltpu.VMEM((_BLK, 1, k), jnp.int32))
    scratch.append(pltpu.VMEM((_BLK, 1, k), jnp.int32))
    scratch.append(pltpu.VMEM_SHARED((n_rows // 8, 128), jnp.float32))

  def body(table_hbm, *rest):
    if with_deg:
      (src_hbm, dst_hbm, div_hbm, pat_hbm, ptab_hbm, sums_out, deg_out,
       src_b, dst_b, rows_v, sums_sh, sem_g, sem_s, div_b, pat_b,
       deg_sh) = rest
    else:
      (src_hbm, dst_hbm, sums_out, src_b, dst_b, rows_v, sums_sh, sem_g,
       sem_s) = rest
    c = lax.axis_index("c")
    s = lax.axis_index("s")
    wid = c * _NS + s

    def zrow(i, _):
      for cc in range(width // 16):
        rows_v[0, i, pl.ds(cc * 16, 16)] = jnp.zeros((16,), jnp.float32)
      return 0
    lax.fori_loop(0, k, zrow, 0)
    base = s * npt
    for t in range(npt // k):
      pltpu.sync_copy(rows_v.at[0], sums_sh.at[pl.ds(base + t * k, k)])
    if with_deg:
      pltpu.sync_copy(rows_v.at[0], deg_sh.at[pl.ds(s * dpt, dpt)])
    plsc.subcore_barrier()

    def run_pipe(mk_g, mk_s):
      mk_g(0, 0).start()

      def step(j, _):
        @pl.when(j >= 1)
        def _():
          mk_s(j - 1, (j - 1) % 2).wait()

        @pl.when(j < _BLK - 1)
        def _():
          mk_g(j + 1, (j + 1) % 2).start()
        mk_g(j, j % 2).wait()
        mk_s(j, j % 2).start(add=True)
        return 0
      lax.fori_loop(0, _BLK, step, 0)
      mk_s(_BLK - 1, (_BLK - 1) % 2).wait()

    def mk_sum_g(j, b):
      return pltpu.make_async_copy(
          table_hbm.at[src_b.at[j, 0]], rows_v.at[b], sem_g)

    def mk_sum_s(j, b):
      return pltpu.make_async_copy(
          rows_v.at[b], sums_sh.at[dst_b.at[j, 0]], sem_s)

    def block_step(bi, _):
      sl = pl.ds(bi * _BLK, _BLK)
      pltpu.sync_copy(src_hbm.at[wid, sl], src_b)
      pltpu.sync_copy(dst_hbm.at[wid, sl], dst_b)
      run_pipe(mk_sum_g, mk_sum_s)
      if with_deg:
        def mk_deg_g(j, b):
          return pltpu.make_async_copy(
              ptab_hbm.at[pat_b.at[j, 0]], rows_v.at[b], sem_g)

        def mk_deg_s(j, b):
          return pltpu.make_async_copy(
              rows_v.at[b], deg_sh.at[div_b.at[j, 0]], sem_s)
        pltpu.sync_copy(div_hbm.at[wid, sl], div_b)
        pltpu.sync_copy(pat_hbm.at[wid, sl], pat_b)
        run_pipe(mk_deg_g, mk_deg_s)
      return 0
    lax.fori_loop(0, nblk, block_step, 0)

    plsc.subcore_barrier()
    pltpu.sync_copy(sums_sh.at[pl.ds(base, npt)],
                    sums_out.at[c, pl.ds(base, npt)])
    if with_deg:
      pltpu.sync_copy(deg_sh.at[pl.ds(s * dpt, dpt)],
                      deg_out.at[c, pl.ds(s * dpt, dpt)])

  fn = pl.kernel(body, out_type=out_type, mesh=mesh, scratch_types=scratch)
  if with_deg:
    return fn(table, srcs, dsts, divs, pats, pattern)
  return fn(table, srcs, dsts)


def _unfold_deg(dp):
  return jnp.maximum(dp[0, :, :1] + dp[1, :, :1], 1.0)


def _tc_layer1(features, sums_p, deg_p, W_self1, W_neigh1, b1,
               W_self2, b2):
  n, d = features.shape
  h = W_self1.shape[1]
  c_dim = W_self2.shape[1]
  r = 1024
  nb = (n + r - 1) // r

  def body(f_ref, sp_ref, dp_ref, ws1_ref, wn1_ref, b1_ref, ws2_ref,
           b2_ref, h1_ref, pre2_ref):
    deg = _unfold_deg(dp_ref[...])
    sp = sp_ref[...]
    sums1 = sp[0] + sp[1]
    hn1 = jnp.dot(sums1, wn1_ref[...],
                  preferred_element_type=jnp.float32) / deg
    h1 = jnp.maximum(
        jnp.dot(f_ref[...], ws1_ref[...], preferred_element_type=jnp.float32)
        + hn1 + b1_ref[...], 0.0)
    h1_ref[...] = h1
    pre2_ref[...] = (
        jnp.dot(h1, ws2_ref[...], preferred_element_type=jnp.float32)
        + b2_ref[...])

  return pl.pallas_call(
      body,
      grid=(nb,),
      in_specs=[
          pl.BlockSpec((r, d), lambda i: (i, 0)),
          pl.BlockSpec((2, r, d), lambda i: (0, i, 0)),
          pl.BlockSpec((2, r, 16), lambda i: (0, i, 0)),
          pl.BlockSpec((d, h), lambda i: (0, 0)),
          pl.BlockSpec((d, h), lambda i: (0, 0)),
          pl.BlockSpec((1, h), lambda i: (0, 0)),
          pl.BlockSpec((h, c_dim), lambda i: (0, 0)),
          pl.BlockSpec((1, c_dim), lambda i: (0, 0)),
      ],
      out_specs=[
          pl.BlockSpec((r, h), lambda i: (i, 0)),
          pl.BlockSpec((r, c_dim), lambda i: (i, 0)),
      ],
      out_shape=[
          jax.ShapeDtypeStruct((n, h), jnp.float32),
          jax.ShapeDtypeStruct((n, c_dim), jnp.float32),
      ],
  )(features, sums_p, deg_p, W_self1, W_neigh1, b1.reshape(1, h),
    W_self2, b2.reshape(1, c_dim))


def _tc_layer2(pre2, sums2_p, deg_p, W_neigh2):
  n, c_dim = pre2.shape
  h = W_neigh2.shape[0]
  r = 1024
  nb = (n + r - 1) // r

  def body(pre_ref, q_ref, dp_ref, wn2_ref, out_ref):
    deg = _unfold_deg(dp_ref[...])
    q = q_ref[...]
    out_ref[...] = pre_ref[...] + jnp.dot(
        q[0] + q[1], wn2_ref[...], preferred_element_type=jnp.float32) / deg

  return pl.pallas_call(
      body,
      grid=(nb,),
      in_specs=[
          pl.BlockSpec((r, c_dim), lambda i: (i, 0)),
          pl.BlockSpec((2, r, h), lambda i: (0, i, 0)),
          pl.BlockSpec((2, r, 16), lambda i: (0, i, 0)),
          pl.BlockSpec((h, c_dim), lambda i: (0, 0)),
      ],
      out_specs=pl.BlockSpec((r, c_dim), lambda i: (i, 0)),
      out_shape=jax.ShapeDtypeStruct((n, c_dim), jnp.float32),
  )(pre2, sums2_p, deg_p, W_neigh2)


@jax.jit
def kernel(features, edge_index, W_self1, W_neigh1, b1, W_self2, W_neigh2,
           b2):
  n = features.shape[0]
  e = edge_index.shape[1]
  ch = e // (_NW * _K)
  src = edge_index[0]
  dst = edge_index[1]
  pattern = jnp.tile(
      jnp.kron(jnp.eye(8, dtype=jnp.float32),
               jnp.ones((1, 16), jnp.float32)), (256, 1))
  pat_idx = (dst & 7) + 8 * (jnp.arange(e, dtype=jnp.int32) & 255)
  shape3 = (_NW, ch, 1, _K)
  srcs = src.reshape(shape3)
  dsts = dst.reshape(shape3)
  divs = (dst >> 3).reshape(shape3)
  pats = pat_idx.reshape(shape3)
  np_rows = -(-n // (_NS * _K)) * (_NS * _K)

  sums1_p, deg_p = _sc_aggregate(features, np_rows, srcs, dsts,
                                 divs=divs, pats=pats, pattern=pattern)
  deg_p = deg_p.reshape(_NC, np_rows, 16)
  h1, pre2 = _tc_layer1(features, sums1_p, deg_p, W_self1, W_neigh1, b1,
                        W_self2, b2)
  (sums2_p,) = _sc_aggregate(h1, np_rows, srcs, dsts)
  return _tc_layer2(pre2, sums2_p, deg_p, W_neigh2)
